# Initial kernel scaffold; baseline (speedup 1.0000x reference)
import jax
import jax.numpy as jnp
from jax.experimental import pallas as pl


def kernel(edge_index, edge_type, node_emb, basis1, comp1, root1, bias1, basis2, comp2, root2, bias2):
    raise NotImplementedError("write your pallas kernel here")


# speedup vs baseline: 2.0499x; 2.0499x over previous
import functools

import jax
import jax.numpy as jnp
from jax import lax
from jax.experimental import pallas as pl
from jax.experimental.pallas import tpu as pltpu
from jax.experimental.pallas import tpu_sc as plsc

N = 10000
H = 128
NR = 50
NBASis = 30
E = 320000

NC = 2
NS = 16
NW = NC * NS

E_PER = E // NW
CHUNK = 80
NCHUNK = E_PER // CHUNK

CTAB = 524288
CT_SLICE = CTAB // NS

N_PAD = 10240
ROWS_PER_TILE = N_PAD // NS

_mesh = plsc.VectorSubcoreMesh(
    core_axis_name="c", subcore_axis_name="s", num_cores=NC, num_subcores=NS)


def _wid():
    return lax.axis_index("c") * NS + lax.axis_index("s")


def _zero_vmem(ref, nwords):
    z = jnp.zeros((16,), jnp.float32)

    def body(i, _):
        ref[pl.ds(i * 16, 16)] = z
        return 0

    lax.fori_loop(0, nwords // 16, body, 0)


def _zero_rows(ref, nrows, ncols):
    z = jnp.zeros((16,), jnp.float32)

    def body(r, _):
        for j in range(ncols // 16):
            ref[r, pl.ds(j * 16, 16)] = z
        return 0

    lax.fori_loop(0, nrows, body, 0)


def _counts_body(dst_hbm, rel_hbm, out_hbm, dst_v, rel_v, key_v, ones_v,
                 zbuf_v, cnt_sp, sem):
    core = lax.axis_index("c")
    sid = lax.axis_index("s")
    wid = _wid()

    _zero_vmem(zbuf_v, 4096)
    for z in range(CT_SLICE // 4096):
        pltpu.sync_copy(zbuf_v, cnt_sp.at[pl.ds(sid * CT_SLICE + z * 4096, 4096)])

    def ones_body(i, _):
        ones_v[pl.ds(i * 16, 16)] = jnp.ones((16,), jnp.float32)
        return 0
    lax.fori_loop(0, CHUNK // 16, ones_body, 0)

    plsc.subcore_barrier()

    def chunk_body(c, _):
        base = pl.multiple_of(wid * E_PER + c * CHUNK, 8)
        pltpu.sync_copy(dst_hbm.at[pl.ds(base, CHUNK)], dst_v)
        pltpu.sync_copy(rel_hbm.at[pl.ds(base, CHUNK)], rel_v)

        def key_body(i, _):
            sl = pl.ds(i * 16, 16)
            key_v[sl] = dst_v[sl] * 50 + rel_v[sl]
            return 0
        lax.fori_loop(0, CHUNK // 16, key_body, 0)

        pltpu.sync_copy(ones_v, cnt_sp.at[key_v], add=True)
        return 0

    lax.fori_loop(0, NCHUNK, chunk_body, 0)

    plsc.subcore_barrier()

    pltpu.sync_copy(cnt_sp.at[pl.ds(sid * CT_SLICE, CT_SLICE)],
                    out_hbm.at[pl.ds(core * CTAB + sid * CT_SLICE, CT_SLICE)])


_counts_call = pl.kernel(
    _counts_body,
    out_type=jax.ShapeDtypeStruct((NC * CTAB,), jnp.float32),
    mesh=_mesh,
    scratch_types=[
        pltpu.VMEM((CHUNK,), jnp.int32),
        pltpu.VMEM((CHUNK,), jnp.int32),
        pltpu.VMEM((CHUNK,), jnp.int32),
        pltpu.VMEM((CHUNK,), jnp.float32),
        pltpu.VMEM((4096,), jnp.float32),
        pltpu.VMEM_SHARED((CTAB,), jnp.float32),
        pltpu.SemaphoreType.DMA,
    ],
)


def _norm_body(dst_hbm, rel_hbm, p0_hbm, p1_hbm, out_hbm, dst_v, rel_v,
               key_v, c0_v, c1_v, norm_v, sem):
    wid = _wid()

    def chunk_body(c, _):
        base = pl.multiple_of(wid * E_PER + c * CHUNK, 8)
        pltpu.sync_copy(dst_hbm.at[pl.ds(base, CHUNK)], dst_v)
        pltpu.sync_copy(rel_hbm.at[pl.ds(base, CHUNK)], rel_v)

        def key_body(i, _):
            sl = pl.ds(i * 16, 16)
            key_v[sl] = dst_v[sl] * 50 + rel_v[sl]
            return 0
        lax.fori_loop(0, CHUNK // 16, key_body, 0)

        pltpu.async_copy(p0_hbm.at[key_v], c0_v, sem).wait()
        pltpu.async_copy(p1_hbm.at[key_v], c1_v, sem).wait()

        def norm_body(i, _):
            sl = pl.ds(i * 16, 16)
            cnt = c0_v[sl] + c1_v[sl]
            norm_v[sl] = 1.0 / jnp.maximum(cnt, 1.0)
            return 0
        lax.fori_loop(0, CHUNK // 16, norm_body, 0)

        pltpu.sync_copy(norm_v, out_hbm.at[pl.ds(base, CHUNK)])
        return 0

    lax.fori_loop(0, NCHUNK, chunk_body, 0)


_norm_call = pl.kernel(
    _norm_body,
    out_type=jax.ShapeDtypeStruct((E,), jnp.float32),
    mesh=_mesh,
    scratch_types=[
        pltpu.VMEM((CHUNK,), jnp.int32),
        pltpu.VMEM((CHUNK,), jnp.int32),
        pltpu.VMEM((CHUNK,), jnp.int32),
        pltpu.VMEM((CHUNK,), jnp.float32),
        pltpu.VMEM((CHUNK,), jnp.float32),
        pltpu.VMEM((CHUNK,), jnp.float32),
        pltpu.SemaphoreType.DMA,
    ],
)


def _agg_body(h_hbm, src_hbm, rel_hbm, dst_hbm, norm_hbm, out_hbm,
              src_v, rel_v, dst_v, idx_v, norm_v, rows_v, acc_sp, sem):
    core = lax.axis_index("c")
    sid = lax.axis_index("s")
    wid = _wid()

    _zero_rows(rows_v, CHUNK, H)
    r0 = sid * ROWS_PER_TILE
    off = 0
    while off < ROWS_PER_TILE:
        nn = min(CHUNK, ROWS_PER_TILE - off)
        pltpu.sync_copy(rows_v.at[pl.ds(0, nn), :],
                        acc_sp.at[pl.ds(r0 + off, nn), :])
        off += nn

    plsc.subcore_barrier()

    def chunk_body(c, _):
        base = pl.multiple_of(wid * E_PER + c * CHUNK, 8)
        pltpu.sync_copy(src_hbm.at[pl.ds(base, CHUNK)], src_v)
        pltpu.sync_copy(rel_hbm.at[pl.ds(base, CHUNK)], rel_v)
        pltpu.sync_copy(dst_hbm.at[pl.ds(base, CHUNK)], dst_v)
        pltpu.sync_copy(norm_hbm.at[pl.ds(base, CHUNK)], norm_v)

        def idx_body(i, _):
            sl = pl.ds(i * 16, 16)
            idx_v[sl] = rel_v[sl] * N + src_v[sl]
            return 0
        lax.fori_loop(0, CHUNK // 16, idx_body, 0)

        pltpu.async_copy(h_hbm.at[idx_v], rows_v, sem).wait()

        def mul_body(g, _):
            nrm = norm_v[pl.ds(g * 16, 16)]
            for l in range(16):
                nb = jnp.full((16,), nrm[l], jnp.float32)
                e = g * 16 + l
                for j in range(H // 16):
                    sl = pl.ds(j * 16, 16)
                    rows_v[e, sl] = rows_v[e, sl] * nb
            return 0
        lax.fori_loop(0, CHUNK // 16, mul_body, 0)

        pltpu.sync_copy(rows_v, acc_sp.at[dst_v], add=True)
        return 0

    lax.fori_loop(0, NCHUNK, chunk_body, 0)

    plsc.subcore_barrier()

    pltpu.sync_copy(acc_sp.at[pl.ds(r0, ROWS_PER_TILE), :],
                    out_hbm.at[pl.ds(core * N_PAD + r0, ROWS_PER_TILE), :])


_agg_call = pl.kernel(
    _agg_body,
    out_type=jax.ShapeDtypeStruct((NC * N_PAD, H), jnp.float32),
    mesh=_mesh,
    scratch_types=[
        pltpu.VMEM((CHUNK,), jnp.int32),
        pltpu.VMEM((CHUNK,), jnp.int32),
        pltpu.VMEM((CHUNK,), jnp.int32),
        pltpu.VMEM((CHUNK,), jnp.int32),
        pltpu.VMEM((CHUNK,), jnp.float32),
        pltpu.VMEM((CHUNK, H), jnp.float32),
        pltpu.VMEM_SHARED((N_PAD, H), jnp.float32),
        pltpu.SemaphoreType.DMA,
    ],
)


NBLK = 400
NGRID = N // NBLK


def _transform_body(x_ref, comp_ref, basis_ref, out_ref):
    w_flat = jnp.dot(comp_ref[...], basis_ref[...],
                     preferred_element_type=jnp.float32)
    x = x_ref[...]
    for r in range(NR):
        w = w_flat[r].reshape(H, H)
        out_ref[r] = jnp.dot(x, w, preferred_element_type=jnp.float32)


def _transform(x, comp, basis_flat):
    return pl.pallas_call(
        _transform_body,
        grid=(NGRID,),
        in_specs=[
            pl.BlockSpec((NBLK, H), lambda i: (i, 0)),
            pl.BlockSpec((NR, NBASis), lambda i: (0, 0)),
            pl.BlockSpec((NBASis, H * H), lambda i: (0, 0)),
        ],
        out_specs=pl.BlockSpec((NR, NBLK, H), lambda i: (0, i, 0)),
        out_shape=jax.ShapeDtypeStruct((NR, N, H), jnp.float32),
    )(x, comp, basis_flat)


def _combine_body(parts_ref, x_ref, root_ref, bias_ref, out_ref, *, relu):
    s = parts_ref[0] + parts_ref[1]
    y = s + jnp.dot(x_ref[...], root_ref[...],
                    preferred_element_type=jnp.float32) + bias_ref[...]
    out_ref[...] = jnp.maximum(y, 0.0) if relu else y


def _combine(parts, x, root, bias2d, relu):
    return pl.pallas_call(
        functools.partial(_combine_body, relu=relu),
        grid=(NGRID,),
        in_specs=[
            pl.BlockSpec((2, NBLK, H), lambda i: (0, i, 0)),
            pl.BlockSpec((NBLK, H), lambda i: (i, 0)),
            pl.BlockSpec((H, H), lambda i: (0, 0)),
            pl.BlockSpec((1, H), lambda i: (0, 0)),
        ],
        out_specs=pl.BlockSpec((NBLK, H), lambda i: (i, 0)),
        out_shape=jax.ShapeDtypeStruct((N, H), jnp.float32),
    )(parts, x, root, bias2d)


def _layer(x, src, rel, dst, norm, basis_flat, comp, root, bias2d, relu):
    h = _transform(x, comp, basis_flat)
    h_flat = h.reshape(NR * N, H)
    parts = _agg_call(h_flat, src, rel, dst, norm)
    return _combine(parts.reshape(NC, N_PAD, H), x, root, bias2d, relu)


def kernel(edge_index, edge_type, node_emb, basis1, comp1, root1, bias1,
           basis2, comp2, root2, bias2):
    src = edge_index[0]
    dst = edge_index[1]
    rel = edge_type

    cnt_parts = _counts_call(dst, rel)
    p0 = cnt_parts[:CTAB]
    p1 = cnt_parts[CTAB:]
    norm = _norm_call(dst, rel, p0, p1)

    b1f = basis1.reshape(NBASis, H * H)
    b2f = basis2.reshape(NBASis, H * H)
    x1 = _layer(node_emb, src, rel, dst, norm, b1f, comp1, root1,
                bias1.reshape(1, H), relu=True)
    out = _layer(x1, src, rel, dst, norm, b2f, comp2, root2,
                 bias2.reshape(1, H), relu=False)
    return out


# speedup vs baseline: 4.1691x; 2.0338x over previous
import functools

import jax
import jax.numpy as jnp
from jax import lax
from jax.experimental import pallas as pl
from jax.experimental.pallas import tpu as pltpu
from jax.experimental.pallas import tpu_sc as plsc

N = 10000
H = 128
NR = 50
NBASis = 30
E = 320000

NC = 2
NS = 16
NW = NC * NS

E_PER = E // NW
CHUNK = 80
NCHUNK = E_PER // CHUNK

CTAB = 524288
CT_SLICE = CTAB // NS

N_PAD = 10240
ROWS_PER_TILE = N_PAD // NS

_mesh = plsc.VectorSubcoreMesh(
    core_axis_name="c", subcore_axis_name="s", num_cores=NC, num_subcores=NS)


def _wid():
    return lax.axis_index("c") * NS + lax.axis_index("s")


def _zero_vmem(ref, nwords):
    z = jnp.zeros((16,), jnp.float32)

    def body(i, _):
        ref[pl.ds(i * 16, 16)] = z
        return 0

    lax.fori_loop(0, nwords // 16, body, 0)


def _zero_rows(ref, nrows, ncols):
    z = jnp.zeros((16,), jnp.float32)

    def body(r, _):
        for j in range(ncols // 16):
            ref[r, pl.ds(j * 16, 16)] = z
        return 0

    lax.fori_loop(0, nrows, body, 0)


def _counts_body(dst_hbm, rel_hbm, out_hbm, dst_v, rel_v, key_v, ones_v,
                 zbuf_v, cnt_sp, sem):
    core = lax.axis_index("c")
    sid = lax.axis_index("s")
    wid = _wid()

    _zero_vmem(zbuf_v, 4096)
    for z in range(CT_SLICE // 4096):
        pltpu.sync_copy(zbuf_v, cnt_sp.at[pl.ds(sid * CT_SLICE + z * 4096, 4096)])

    def ones_body(i, _):
        ones_v[pl.ds(i * 16, 16)] = jnp.ones((16,), jnp.float32)
        return 0
    lax.fori_loop(0, CHUNK // 16, ones_body, 0)

    plsc.subcore_barrier()

    def chunk_body(c, _):
        base = pl.multiple_of(wid * E_PER + c * CHUNK, 8)
        pltpu.sync_copy(dst_hbm.at[pl.ds(base, CHUNK)], dst_v)
        pltpu.sync_copy(rel_hbm.at[pl.ds(base, CHUNK)], rel_v)

        def key_body(i, _):
            sl = pl.ds(i * 16, 16)
            key_v[sl] = dst_v[sl] * 50 + rel_v[sl]
            return 0
        lax.fori_loop(0, CHUNK // 16, key_body, 0)

        pltpu.sync_copy(ones_v, cnt_sp.at[key_v], add=True)
        return 0

    lax.fori_loop(0, NCHUNK, chunk_body, 0)

    plsc.subcore_barrier()

    pltpu.sync_copy(cnt_sp.at[pl.ds(sid * CT_SLICE, CT_SLICE)],
                    out_hbm.at[pl.ds(core * CTAB + sid * CT_SLICE, CT_SLICE)])


_counts_call = pl.kernel(
    _counts_body,
    out_type=jax.ShapeDtypeStruct((NC * CTAB,), jnp.float32),
    mesh=_mesh,
    scratch_types=[
        pltpu.VMEM((CHUNK,), jnp.int32),
        pltpu.VMEM((CHUNK,), jnp.int32),
        pltpu.VMEM((CHUNK,), jnp.int32),
        pltpu.VMEM((CHUNK,), jnp.float32),
        pltpu.VMEM((4096,), jnp.float32),
        pltpu.VMEM_SHARED((CTAB,), jnp.float32),
        pltpu.SemaphoreType.DMA,
    ],
)


CT_ROWS = CTAB // H
CT_BLK = 512


def _ntab_body(p_ref, out_ref):
    out_ref[...] = 1.0 / jnp.maximum(p_ref[0] + p_ref[1], 1.0)


def _ntab(cnt_parts):
    t = pl.pallas_call(
        _ntab_body,
        grid=(CT_ROWS // CT_BLK,),
        in_specs=[pl.BlockSpec((2, CT_BLK, H), lambda i: (0, i, 0))],
        out_specs=pl.BlockSpec((CT_BLK, H), lambda i: (i, 0)),
        out_shape=jax.ShapeDtypeStruct((CT_ROWS, H), jnp.float32),
    )(cnt_parts.reshape(2, CT_ROWS, H))
    return t.reshape(CTAB)


SUB = 80
NSUB = 4
SUP = SUB * NSUB
NSUP = E_PER // SUP
TAIL = E_PER - NSUP * SUP


def _agg_body(h_hbm, src_hbm, rel_hbm, dst_hbm, ntab_hbm, out_hbm,
              src_v, rel_v, dst_v, idx2_v, key2_v, dst2_v, norm_v, rows_v,
              acc_sp, sem):
    core = lax.axis_index("c")
    sid = lax.axis_index("s")
    wid = _wid()

    _zero_rows(rows_v, SUP, H)
    r0 = sid * ROWS_PER_TILE
    off = 0
    while off < ROWS_PER_TILE:
        nn = min(SUP, ROWS_PER_TILE - off)
        pltpu.sync_copy(rows_v.at[pl.ds(0, nn), :],
                        acc_sp.at[pl.ds(r0 + off, nn), :])
        off += nn

    plsc.subcore_barrier()

    def process(base, nsub):
        n_e = nsub * SUB
        cps = [pltpu.async_copy(src_hbm.at[pl.ds(base, n_e)],
                                src_v.at[pl.ds(0, n_e)], sem),
               pltpu.async_copy(rel_hbm.at[pl.ds(base, n_e)],
                                rel_v.at[pl.ds(0, n_e)], sem),
               pltpu.async_copy(dst_hbm.at[pl.ds(base, n_e)],
                                dst_v.at[pl.ds(0, n_e)], sem)]
        for cp in cps:
            cp.wait()

        def idx_body(g, _):
            j = g // (SUB // 16)
            col = (g % (SUB // 16)) * 16
            sl = pl.ds(g * 16, 16)
            csl = pl.ds(col, 16)
            idx2_v[j, csl] = rel_v[sl] * N + src_v[sl]
            key2_v[j, csl] = dst_v[sl] * 50 + rel_v[sl]
            dst2_v[j, csl] = dst_v[sl]
            return 0
        lax.fori_loop(0, n_e // 16, idx_body, 0)

        cps = []
        for j in range(nsub):
            cps.append(pltpu.async_copy(
                h_hbm.at[idx2_v.at[j]], rows_v.at[pl.ds(j * SUB, SUB), :], sem))
            cps.append(pltpu.async_copy(
                ntab_hbm.at[key2_v.at[j]], norm_v.at[pl.ds(j * SUB, SUB)], sem))
        for cp in cps:
            cp.wait()

        def mul_body(g, _):
            nrm = norm_v[pl.ds(g * 16, 16)]
            for l in range(16):
                nb = jnp.full((16,), nrm[l], jnp.float32)
                e = g * 16 + l
                for j in range(H // 16):
                    sl = pl.ds(j * 16, 16)
                    rows_v[e, sl] = rows_v[e, sl] * nb
            return 0
        lax.fori_loop(0, n_e // 16, mul_body, 0)

        cps = [pltpu.async_copy(rows_v.at[pl.ds(j * SUB, SUB), :],
                                acc_sp.at[dst2_v.at[j]], sem, add=True)
               for j in range(nsub)]
        for cp in cps:
            cp.wait()

    def sup_body(c, _):
        process(pl.multiple_of(wid * E_PER + c * SUP, 8), NSUB)
        return 0

    lax.fori_loop(0, NSUP, sup_body, 0)
    if TAIL:
        process(pl.multiple_of(wid * E_PER + NSUP * SUP, 8), TAIL // SUB)

    plsc.subcore_barrier()

    pltpu.sync_copy(acc_sp.at[pl.ds(r0, ROWS_PER_TILE), :],
                    out_hbm.at[pl.ds(core * N_PAD + r0, ROWS_PER_TILE), :])


_agg_call = pl.kernel(
    _agg_body,
    out_type=jax.ShapeDtypeStruct((NC * N_PAD, H), jnp.float32),
    mesh=_mesh,
    scratch_types=[
        pltpu.VMEM((SUP,), jnp.int32),
        pltpu.VMEM((SUP,), jnp.int32),
        pltpu.VMEM((SUP,), jnp.int32),
        pltpu.VMEM((NSUB, SUB), jnp.int32),
        pltpu.VMEM((NSUB, SUB), jnp.int32),
        pltpu.VMEM((NSUB, SUB), jnp.int32),
        pltpu.VMEM((SUP,), jnp.float32),
        pltpu.VMEM((SUP, H), jnp.float32),
        pltpu.VMEM_SHARED((N_PAD, H), jnp.float32),
        pltpu.SemaphoreType.DMA,
    ],
)


NBLK = 400
NGRID = N // NBLK


def _transform_body(x_ref, comp_ref, basis_ref, out_ref):
    w_flat = jnp.dot(comp_ref[...], basis_ref[...],
                     preferred_element_type=jnp.float32)
    x = x_ref[...]
    for r in range(NR):
        w = w_flat[r].reshape(H, H)
        out_ref[r] = jnp.dot(x, w, preferred_element_type=jnp.float32)


def _transform(x, comp, basis_flat):
    return pl.pallas_call(
        _transform_body,
        grid=(NGRID,),
        in_specs=[
            pl.BlockSpec((NBLK, H), lambda i: (i, 0)),
            pl.BlockSpec((NR, NBASis), lambda i: (0, 0)),
            pl.BlockSpec((NBASis, H * H), lambda i: (0, 0)),
        ],
        out_specs=pl.BlockSpec((NR, NBLK, H), lambda i: (0, i, 0)),
        out_shape=jax.ShapeDtypeStruct((NR, N, H), jnp.float32),
    )(x, comp, basis_flat)


def _combine_body(parts_ref, x_ref, root_ref, bias_ref, out_ref, *, relu):
    s = parts_ref[0] + parts_ref[1]
    y = s + jnp.dot(x_ref[...], root_ref[...],
                    preferred_element_type=jnp.float32) + bias_ref[...]
    out_ref[...] = jnp.maximum(y, 0.0) if relu else y


def _combine(parts, x, root, bias2d, relu):
    return pl.pallas_call(
        functools.partial(_combine_body, relu=relu),
        grid=(NGRID,),
        in_specs=[
            pl.BlockSpec((2, NBLK, H), lambda i: (0, i, 0)),
            pl.BlockSpec((NBLK, H), lambda i: (i, 0)),
            pl.BlockSpec((H, H), lambda i: (0, 0)),
            pl.BlockSpec((1, H), lambda i: (0, 0)),
        ],
        out_specs=pl.BlockSpec((NBLK, H), lambda i: (i, 0)),
        out_shape=jax.ShapeDtypeStruct((N, H), jnp.float32),
    )(parts, x, root, bias2d)


def _layer(x, src, rel, dst, ntab, basis_flat, comp, root, bias2d, relu):
    h = _transform(x, comp, basis_flat)
    h_flat = h.reshape(NR * N, H)
    parts = _agg_call(h_flat, src, rel, dst, ntab)
    return _combine(parts.reshape(NC, N_PAD, H), x, root, bias2d, relu)


def kernel(edge_index, edge_type, node_emb, basis1, comp1, root1, bias1,
           basis2, comp2, root2, bias2):
    src = edge_index[0]
    dst = edge_index[1]
    rel = edge_type

    cnt_parts = _counts_call(dst, rel)
    ntab = _ntab(cnt_parts)

    b1f = basis1.reshape(NBASis, H * H)
    b2f = basis2.reshape(NBASis, H * H)
    x1 = _layer(node_emb, src, rel, dst, ntab, b1f, comp1, root1,
                bias1.reshape(1, H), relu=True)
    out = _layer(x1, src, rel, dst, ntab, b2f, comp2, root2,
                 bias2.reshape(1, H), relu=False)
    return out


# speedup vs baseline: 4.8767x; 1.1697x over previous
import functools

import jax
import jax.numpy as jnp
from jax import lax
from jax.experimental import pallas as pl
from jax.experimental.pallas import tpu as pltpu
from jax.experimental.pallas import tpu_sc as plsc

N = 10000
H = 128
NR = 50
NBASis = 30
E = 320000

NC = 2
NS = 16
NW = NC * NS

E_PER = E // NW
CHUNK = 80
NCHUNK = E_PER // CHUNK

CTAB = 524288
CT_SLICE = CTAB // NS

N_PAD = 10240
ROWS_PER_TILE = N_PAD // NS

SUB = 80
NSUB = 4
SUP = SUB * NSUB
NSUP = E_PER // SUP
TAIL = E_PER - NSUP * SUP

_mesh = plsc.VectorSubcoreMesh(
    core_axis_name="c", subcore_axis_name="s", num_cores=NC, num_subcores=NS)


def _wid():
    return lax.axis_index("c") * NS + lax.axis_index("s")


def _zero_vmem(ref, nwords):
    z = jnp.zeros((16,), jnp.float32)

    def body(i, _):
        ref[pl.ds(i * 16, 16)] = z
        return 0

    lax.fori_loop(0, nwords // 16, body, 0)


def _zero_rows(ref, nrows, ncols):
    z = jnp.zeros((16,), jnp.float32)

    def body(r, _):
        for j in range(ncols // 16):
            ref[r, pl.ds(j * 16, 16)] = z
        return 0

    lax.fori_loop(0, nrows, body, 0)


def _counts_body(dst_hbm, rel_hbm, out_hbm, dst_v, rel_v, key2_v, ones_v,
                 zbuf_v, cnt_sp, sem_e, sem_s):
    core = lax.axis_index("c")
    sid = lax.axis_index("s")
    wid = _wid()

    _zero_vmem(zbuf_v, 4096)
    for z in range(CT_SLICE // 4096):
        pltpu.sync_copy(zbuf_v, cnt_sp.at[pl.ds(sid * CT_SLICE + z * 4096, 4096)])

    def ones_body(i, _):
        ones_v[pl.ds(i * 16, 16)] = jnp.ones((16,), jnp.float32)
        return 0
    lax.fori_loop(0, SUB // 16, ones_body, 0)

    plsc.subcore_barrier()

    def process(c, base, nsub):
        n_e = nsub * SUB
        cps = [pltpu.async_copy(dst_hbm.at[pl.ds(base, n_e)],
                                dst_v.at[pl.ds(0, n_e)], sem_e),
               pltpu.async_copy(rel_hbm.at[pl.ds(base, n_e)],
                                rel_v.at[pl.ds(0, n_e)], sem_e)]
        for cp in cps:
            cp.wait()

        def key_body(g, _):
            j = g // (SUB // 16)
            col = (g % (SUB // 16)) * 16
            sl = pl.ds(g * 16, 16)
            key2_v[j, pl.ds(col, 16)] = dst_v[sl] * 50 + rel_v[sl]
            return 0
        lax.fori_loop(0, n_e // 16, key_body, 0)

        cps = [pltpu.async_copy(ones_v, cnt_sp.at[key2_v.at[j]], sem_s, add=True)
               for j in range(nsub)]
        for cp in cps:
            cp.wait()

    def sup_body(c, _):
        process(c, pl.multiple_of(wid * E_PER + c * SUP, 8), NSUB)
        return 0

    lax.fori_loop(0, NSUP, sup_body, 0)
    if TAIL:
        process(NSUP, pl.multiple_of(wid * E_PER + NSUP * SUP, 8), TAIL // SUB)

    plsc.subcore_barrier()

    pltpu.sync_copy(cnt_sp.at[pl.ds(sid * CT_SLICE, CT_SLICE)],
                    out_hbm.at[pl.ds(core * CTAB + sid * CT_SLICE, CT_SLICE)])


_counts_call = pl.kernel(
    _counts_body,
    out_type=jax.ShapeDtypeStruct((NC * CTAB,), jnp.float32),
    mesh=_mesh,
    scratch_types=[
        pltpu.VMEM((SUP,), jnp.int32),
        pltpu.VMEM((SUP,), jnp.int32),
        pltpu.VMEM((NSUB, SUB), jnp.int32),
        pltpu.VMEM((SUB,), jnp.float32),
        pltpu.VMEM((4096,), jnp.float32),
        pltpu.VMEM_SHARED((CTAB,), jnp.float32),
        pltpu.SemaphoreType.DMA,
        pltpu.SemaphoreType.DMA,
    ],
)


CT_ROWS = CTAB // H
CT_BLK = 512


def _ntab_body(p_ref, out_ref):
    out_ref[...] = 1.0 / jnp.maximum(p_ref[0] + p_ref[1], 1.0)


def _ntab(cnt_parts):
    t = pl.pallas_call(
        _ntab_body,
        grid=(CT_ROWS // CT_BLK,),
        in_specs=[pl.BlockSpec((2, CT_BLK, H), lambda i: (0, i, 0))],
        out_specs=pl.BlockSpec((CT_BLK, H), lambda i: (i, 0)),
        out_shape=jax.ShapeDtypeStruct((CT_ROWS, H), jnp.float32),
    )(cnt_parts.reshape(2, CT_ROWS, H))
    return t.reshape(CTAB)


def _agg_body(h_hbm, src_hbm, rel_hbm, dst_hbm, ntab_hbm, out_hbm,
              src_v, rel_v, dst_v, idx2_v, key2_v, dst2_v, norm_v, rows_v,
              acc_sp, sem_e, sem_s, sem_j0, sem_j1, sem_j2, sem_j3):
    sems = (sem_j0, sem_j1, sem_j2, sem_j3)
    core = lax.axis_index("c")
    sid = lax.axis_index("s")
    wid = _wid()

    _zero_rows(rows_v, SUP, H)
    r0 = sid * ROWS_PER_TILE
    off = 0
    while off < ROWS_PER_TILE:
        nn = min(SUP, ROWS_PER_TILE - off)
        pltpu.sync_copy(rows_v.at[pl.ds(0, nn), :],
                        acc_sp.at[pl.ds(r0 + off, nn), :])
        off += nn

    plsc.subcore_barrier()

    def process(c, base, nsub):
        n_e = nsub * SUB
        cps = [pltpu.async_copy(src_hbm.at[pl.ds(base, n_e)],
                                src_v.at[pl.ds(0, n_e)], sem_e),
               pltpu.async_copy(rel_hbm.at[pl.ds(base, n_e)],
                                rel_v.at[pl.ds(0, n_e)], sem_e),
               pltpu.async_copy(dst_hbm.at[pl.ds(base, n_e)],
                                dst_v.at[pl.ds(0, n_e)], sem_e)]
        for cp in cps:
            cp.wait()

        def idx_body(g, _):
            j = g // (SUB // 16)
            col = (g % (SUB // 16)) * 16
            sl = pl.ds(g * 16, 16)
            csl = pl.ds(col, 16)
            idx2_v[j, csl] = rel_v[sl] * N + src_v[sl]
            key2_v[j, csl] = dst_v[sl] * 50 + rel_v[sl]
            dst2_v[j, csl] = dst_v[sl]
            return 0
        lax.fori_loop(0, n_e // 16, idx_body, 0)

        cps = []
        for j in range(nsub):
            cps.append((
                pltpu.async_copy(h_hbm.at[idx2_v.at[j]],
                                 rows_v.at[pl.ds(j * SUB, SUB), :], sems[j]),
                pltpu.async_copy(ntab_hbm.at[key2_v.at[j]],
                                 norm_v.at[pl.ds(j * SUB, SUB)], sems[j])))

        scs = []
        for j in range(nsub):
            cps[j][0].wait()
            cps[j][1].wait()

            def mul_body(g, _):
                nrm = norm_v[pl.ds(j * SUB + g * 16, 16)]
                for l in range(16):
                    nb = jnp.full((16,), nrm[l], jnp.float32)
                    e = j * SUB + g * 16 + l
                    for jj in range(H // 16):
                        sl = pl.ds(jj * 16, 16)
                        rows_v[e, sl] = rows_v[e, sl] * nb
                return 0
            lax.fori_loop(0, SUB // 16, mul_body, 0)

            scs.append(pltpu.async_copy(rows_v.at[pl.ds(j * SUB, SUB), :],
                                        acc_sp.at[dst2_v.at[j]], sem_s,
                                        add=True))
        for sc in scs:
            sc.wait()

    def sup_body(c, _):
        process(c, pl.multiple_of(wid * E_PER + c * SUP, 8), NSUB)
        return 0

    lax.fori_loop(0, NSUP, sup_body, 0)
    if TAIL:
        process(NSUP, pl.multiple_of(wid * E_PER + NSUP * SUP, 8), TAIL // SUB)

    plsc.subcore_barrier()

    pltpu.sync_copy(acc_sp.at[pl.ds(r0, ROWS_PER_TILE), :],
                    out_hbm.at[pl.ds(core * N_PAD + r0, ROWS_PER_TILE), :])


_agg_call = pl.kernel(
    _agg_body,
    out_type=jax.ShapeDtypeStruct((NC * N_PAD, H), jnp.float32),
    mesh=_mesh,
    scratch_types=[
        pltpu.VMEM((SUP,), jnp.int32),
        pltpu.VMEM((SUP,), jnp.int32),
        pltpu.VMEM((SUP,), jnp.int32),
        pltpu.VMEM((NSUB, SUB), jnp.int32),
        pltpu.VMEM((NSUB, SUB), jnp.int32),
        pltpu.VMEM((NSUB, SUB), jnp.int32),
        pltpu.VMEM((SUP,), jnp.float32),
        pltpu.VMEM((SUP, H), jnp.float32),
        pltpu.VMEM_SHARED((N_PAD, H), jnp.float32),
        pltpu.SemaphoreType.DMA,
        pltpu.SemaphoreType.DMA,
        pltpu.SemaphoreType.DMA,
        pltpu.SemaphoreType.DMA,
        pltpu.SemaphoreType.DMA,
        pltpu.SemaphoreType.DMA,
    ],
)


NBLK = 400
NGRID = N // NBLK


def _transform_body(x_ref, comp_ref, basis_ref, out_ref):
    w_flat = jnp.dot(comp_ref[...], basis_ref[...],
                     preferred_element_type=jnp.float32)
    x = x_ref[...]
    for r in range(NR):
        w = w_flat[r].reshape(H, H)
        out_ref[r] = jnp.dot(x, w, preferred_element_type=jnp.float32)


def _transform(x, comp, basis_flat):
    return pl.pallas_call(
        _transform_body,
        grid=(NGRID,),
        in_specs=[
            pl.BlockSpec((NBLK, H), lambda i: (i, 0)),
            pl.BlockSpec((NR, NBASis), lambda i: (0, 0)),
            pl.BlockSpec((NBASis, H * H), lambda i: (0, 0)),
        ],
        out_specs=pl.BlockSpec((NR, NBLK, H), lambda i: (0, i, 0)),
        out_shape=jax.ShapeDtypeStruct((NR, N, H), jnp.float32),
    )(x, comp, basis_flat)


def _combine_body(parts_ref, x_ref, root_ref, bias_ref, out_ref, *, relu):
    s = parts_ref[0] + parts_ref[1]
    y = s + jnp.dot(x_ref[...], root_ref[...],
                    preferred_element_type=jnp.float32) + bias_ref[...]
    out_ref[...] = jnp.maximum(y, 0.0) if relu else y


def _combine(parts, x, root, bias2d, relu):
    return pl.pallas_call(
        functools.partial(_combine_body, relu=relu),
        grid=(NGRID,),
        in_specs=[
            pl.BlockSpec((2, NBLK, H), lambda i: (0, i, 0)),
            pl.BlockSpec((NBLK, H), lambda i: (i, 0)),
            pl.BlockSpec((H, H), lambda i: (0, 0)),
            pl.BlockSpec((1, H), lambda i: (0, 0)),
        ],
        out_specs=pl.BlockSpec((NBLK, H), lambda i: (i, 0)),
        out_shape=jax.ShapeDtypeStruct((N, H), jnp.float32),
    )(parts, x, root, bias2d)


def _layer(x, src, rel, dst, ntab, basis_flat, comp, root, bias2d, relu):
    h = _transform(x, comp, basis_flat)
    h_flat = h.reshape(NR * N, H)
    parts = _agg_call(h_flat, src, rel, dst, ntab)
    return _combine(parts.reshape(NC, N_PAD, H), x, root, bias2d, relu)


def kernel(edge_index, edge_type, node_emb, basis1, comp1, root1, bias1,
           basis2, comp2, root2, bias2):
    src = edge_index[0]
    dst = edge_index[1]
    rel = edge_type

    cnt_parts = _counts_call(dst, rel)
    ntab = _ntab(cnt_parts)

    b1f = basis1.reshape(NBASis, H * H)
    b2f = basis2.reshape(NBASis, H * H)
    x1 = _layer(node_emb, src, rel, dst, ntab, b1f, comp1, root1,
                bias1.reshape(1, H), relu=True)
    out = _layer(x1, src, rel, dst, ntab, b2f, comp2, root2,
                 bias2.reshape(1, H), relu=False)
    return out


# speedup vs baseline: 4.9097x; 1.0068x over previous
import functools

import jax
import jax.numpy as jnp
from jax import lax
from jax.experimental import pallas as pl
from jax.experimental.pallas import tpu as pltpu
from jax.experimental.pallas import tpu_sc as plsc

N = 10000
H = 128
NR = 50
NBASis = 30
E = 320000

NC = 2
NS = 16
NW = NC * NS

E_PER = E // NW
CHUNK = 80
NCHUNK = E_PER // CHUNK

CTAB = 524288
CT_SLICE = CTAB // NS

N_PAD = 10240
ROWS_PER_TILE = N_PAD // NS

SUB = 80
NSUB = 4
SUP = SUB * NSUB
NSUP = E_PER // SUP
TAIL = E_PER - NSUP * SUP

_mesh = plsc.VectorSubcoreMesh(
    core_axis_name="c", subcore_axis_name="s", num_cores=NC, num_subcores=NS)


def _wid():
    return lax.axis_index("c") * NS + lax.axis_index("s")


def _zero_vmem(ref, nwords):
    z = jnp.zeros((16,), jnp.float32)

    def body(i, _):
        ref[pl.ds(i * 16, 16)] = z
        return 0

    lax.fori_loop(0, nwords // 16, body, 0)


def _zero_rows(ref, nrows, ncols):
    z = jnp.zeros((16,), jnp.float32)

    def body(r, _):
        for j in range(ncols // 16):
            ref[r, pl.ds(j * 16, 16)] = z
        return 0

    lax.fori_loop(0, nrows, body, 0)


def _counts_body(dst_hbm, rel_hbm, out_hbm, dst_v, rel_v, key2_v, ones_v,
                 zbuf_v, cnt_sp, sem_e, sem_s):
    core = lax.axis_index("c")
    sid = lax.axis_index("s")
    wid = _wid()

    _zero_vmem(zbuf_v, 4096)
    for z in range(CT_SLICE // 4096):
        pltpu.sync_copy(zbuf_v, cnt_sp.at[pl.ds(sid * CT_SLICE + z * 4096, 4096)])

    def ones_body(i, _):
        ones_v[pl.ds(i * 16, 16)] = jnp.ones((16,), jnp.float32)
        return 0
    lax.fori_loop(0, SUB // 16, ones_body, 0)

    plsc.subcore_barrier()

    def process(c, base, nsub):
        n_e = nsub * SUB
        cps = [pltpu.async_copy(dst_hbm.at[pl.ds(base, n_e)],
                                dst_v.at[pl.ds(0, n_e)], sem_e),
               pltpu.async_copy(rel_hbm.at[pl.ds(base, n_e)],
                                rel_v.at[pl.ds(0, n_e)], sem_e)]
        for cp in cps:
            cp.wait()

        def key_body(g, _):
            j = g // (SUB // 16)
            col = (g % (SUB // 16)) * 16
            sl = pl.ds(g * 16, 16)
            key2_v[j, pl.ds(col, 16)] = dst_v[sl] * 50 + rel_v[sl]
            return 0
        lax.fori_loop(0, n_e // 16, key_body, 0)

        cps = [pltpu.async_copy(ones_v, cnt_sp.at[key2_v.at[j]], sem_s, add=True)
               for j in range(nsub)]
        for cp in cps:
            cp.wait()

    def sup_body(c, _):
        process(c, pl.multiple_of(wid * E_PER + c * SUP, 8), NSUB)
        return 0

    lax.fori_loop(0, NSUP, sup_body, 0)
    if TAIL:
        process(NSUP, pl.multiple_of(wid * E_PER + NSUP * SUP, 8), TAIL // SUB)

    plsc.subcore_barrier()

    pltpu.sync_copy(cnt_sp.at[pl.ds(sid * CT_SLICE, CT_SLICE)],
                    out_hbm.at[pl.ds(core * CTAB + sid * CT_SLICE, CT_SLICE)])


_counts_call = pl.kernel(
    _counts_body,
    out_type=jax.ShapeDtypeStruct((NC * CTAB,), jnp.float32),
    mesh=_mesh,
    scratch_types=[
        pltpu.VMEM((SUP,), jnp.int32),
        pltpu.VMEM((SUP,), jnp.int32),
        pltpu.VMEM((NSUB, SUB), jnp.int32),
        pltpu.VMEM((SUB,), jnp.float32),
        pltpu.VMEM((4096,), jnp.float32),
        pltpu.VMEM_SHARED((CTAB,), jnp.float32),
        pltpu.SemaphoreType.DMA,
        pltpu.SemaphoreType.DMA,
    ],
)


def _agg_body(h_hbm, src_hbm, rel_hbm, dst_hbm, p0_hbm, p1_hbm, out_hbm,
              src_v, rel_v, dst_v, idx2_v, key2_v, dst2_v, c0_v, c1_v, rows_v,
              acc_sp, sem_e, sem_s, sem_j0, sem_j1, sem_j2, sem_j3):
    sems = (sem_j0, sem_j1, sem_j2, sem_j3)
    core = lax.axis_index("c")
    sid = lax.axis_index("s")
    wid = _wid()

    _zero_rows(rows_v, SUP, H)
    r0 = sid * ROWS_PER_TILE
    off = 0
    while off < ROWS_PER_TILE:
        nn = min(SUP, ROWS_PER_TILE - off)
        pltpu.sync_copy(rows_v.at[pl.ds(0, nn), :],
                        acc_sp.at[pl.ds(r0 + off, nn), :])
        off += nn

    plsc.subcore_barrier()

    def process(c, base, nsub):
        n_e = nsub * SUB
        cps = [pltpu.async_copy(src_hbm.at[pl.ds(base, n_e)],
                                src_v.at[pl.ds(0, n_e)], sem_e),
               pltpu.async_copy(rel_hbm.at[pl.ds(base, n_e)],
                                rel_v.at[pl.ds(0, n_e)], sem_e),
               pltpu.async_copy(dst_hbm.at[pl.ds(base, n_e)],
                                dst_v.at[pl.ds(0, n_e)], sem_e)]
        for cp in cps:
            cp.wait()

        def idx_body(g, _):
            j = g // (SUB // 16)
            col = (g % (SUB // 16)) * 16
            sl = pl.ds(g * 16, 16)
            csl = pl.ds(col, 16)
            idx2_v[j, csl] = rel_v[sl] * N + src_v[sl]
            key2_v[j, csl] = dst_v[sl] * 50 + rel_v[sl]
            dst2_v[j, csl] = dst_v[sl]
            return 0
        lax.fori_loop(0, n_e // 16, idx_body, 0)

        cps = []
        for j in range(nsub):
            cps.append((
                pltpu.async_copy(h_hbm.at[idx2_v.at[j]],
                                 rows_v.at[pl.ds(j * SUB, SUB), :], sems[j]),
                pltpu.async_copy(p0_hbm.at[key2_v.at[j]],
                                 c0_v.at[pl.ds(j * SUB, SUB)], sems[j]),
                pltpu.async_copy(p1_hbm.at[key2_v.at[j]],
                                 c1_v.at[pl.ds(j * SUB, SUB)], sems[j])))

        scs = []
        for j in range(nsub):
            for cp in cps[j]:
                cp.wait()

            def mul_body(g, _):
                sl16 = pl.ds(j * SUB + g * 16, 16)
                nrm = 1.0 / jnp.maximum(c0_v[sl16] + c1_v[sl16], 1.0)
                for l in range(16):
                    nb = jnp.full((16,), nrm[l], jnp.float32)
                    e = j * SUB + g * 16 + l
                    for jj in range(H // 16):
                        sl = pl.ds(jj * 16, 16)
                        rows_v[e, sl] = rows_v[e, sl] * nb
                return 0
            lax.fori_loop(0, SUB // 16, mul_body, 0)

            scs.append(pltpu.async_copy(rows_v.at[pl.ds(j * SUB, SUB), :],
                                        acc_sp.at[dst2_v.at[j]], sem_s,
                                        add=True))
        for sc in scs:
            sc.wait()

    def sup_body(c, _):
        process(c, pl.multiple_of(wid * E_PER + c * SUP, 8), NSUB)
        return 0

    lax.fori_loop(0, NSUP, sup_body, 0)
    if TAIL:
        process(NSUP, pl.multiple_of(wid * E_PER + NSUP * SUP, 8), TAIL // SUB)

    plsc.subcore_barrier()

    pltpu.sync_copy(acc_sp.at[pl.ds(r0, ROWS_PER_TILE), :],
                    out_hbm.at[pl.ds(core * N_PAD + r0, ROWS_PER_TILE), :])


_agg_call = pl.kernel(
    _agg_body,
    out_type=jax.ShapeDtypeStruct((NC * N_PAD, H), jnp.float32),
    mesh=_mesh,
    scratch_types=[
        pltpu.VMEM((SUP,), jnp.int32),
        pltpu.VMEM((SUP,), jnp.int32),
        pltpu.VMEM((SUP,), jnp.int32),
        pltpu.VMEM((NSUB, SUB), jnp.int32),
        pltpu.VMEM((NSUB, SUB), jnp.int32),
        pltpu.VMEM((NSUB, SUB), jnp.int32),
        pltpu.VMEM((SUP,), jnp.float32),
        pltpu.VMEM((SUP,), jnp.float32),
        pltpu.VMEM((SUP, H), jnp.float32),
        pltpu.VMEM_SHARED((N_PAD, H), jnp.float32),
        pltpu.SemaphoreType.DMA,
        pltpu.SemaphoreType.DMA,
        pltpu.SemaphoreType.DMA,
        pltpu.SemaphoreType.DMA,
        pltpu.SemaphoreType.DMA,
        pltpu.SemaphoreType.DMA,
    ],
)


NBLK = 400
NGRID = N // NBLK


def _transform_body(x_ref, comp_ref, basis_ref, out_ref):
    w_flat = jnp.dot(comp_ref[...], basis_ref[...],
                     preferred_element_type=jnp.float32)
    x = x_ref[...]
    for r in range(NR):
        w = w_flat[r].reshape(H, H)
        out_ref[r] = jnp.dot(x, w, preferred_element_type=jnp.float32)


def _transform(x, comp, basis_flat):
    return pl.pallas_call(
        _transform_body,
        grid=(NGRID,),
        in_specs=[
            pl.BlockSpec((NBLK, H), lambda i: (i, 0)),
            pl.BlockSpec((NR, NBASis), lambda i: (0, 0)),
            pl.BlockSpec((NBASis, H * H), lambda i: (0, 0)),
        ],
        out_specs=pl.BlockSpec((NR, NBLK, H), lambda i: (0, i, 0)),
        out_shape=jax.ShapeDtypeStruct((NR, N, H), jnp.float32),
    )(x, comp, basis_flat)


def _combine_body(parts_ref, x_ref, root_ref, bias_ref, out_ref, *, relu):
    s = parts_ref[0] + parts_ref[1]
    y = s + jnp.dot(x_ref[...], root_ref[...],
                    preferred_element_type=jnp.float32) + bias_ref[...]
    out_ref[...] = jnp.maximum(y, 0.0) if relu else y


def _combine(parts, x, root, bias2d, relu):
    return pl.pallas_call(
        functools.partial(_combine_body, relu=relu),
        grid=(NGRID,),
        in_specs=[
            pl.BlockSpec((2, NBLK, H), lambda i: (0, i, 0)),
            pl.BlockSpec((NBLK, H), lambda i: (i, 0)),
            pl.BlockSpec((H, H), lambda i: (0, 0)),
            pl.BlockSpec((1, H), lambda i: (0, 0)),
        ],
        out_specs=pl.BlockSpec((NBLK, H), lambda i: (i, 0)),
        out_shape=jax.ShapeDtypeStruct((N, H), jnp.float32),
    )(parts, x, root, bias2d)


def _combine_transform_body(parts_ref, x_ref, root_ref, bias_ref, comp_ref,
                            basis_ref, x1_ref, h_ref):
    x1 = jnp.maximum(parts_ref[0] + parts_ref[1]
                     + jnp.dot(x_ref[...], root_ref[...],
                               preferred_element_type=jnp.float32)
                     + bias_ref[...], 0.0)
    x1_ref[...] = x1
    w_flat = jnp.dot(comp_ref[...], basis_ref[...],
                     preferred_element_type=jnp.float32)
    for r in range(NR):
        w = w_flat[r].reshape(H, H)
        h_ref[r] = jnp.dot(x1, w, preferred_element_type=jnp.float32)


def _combine_transform(parts, x, root, bias2d, comp, basis_flat):
    return pl.pallas_call(
        _combine_transform_body,
        grid=(NGRID,),
        in_specs=[
            pl.BlockSpec((2, NBLK, H), lambda i: (0, i, 0)),
            pl.BlockSpec((NBLK, H), lambda i: (i, 0)),
            pl.BlockSpec((H, H), lambda i: (0, 0)),
            pl.BlockSpec((1, H), lambda i: (0, 0)),
            pl.BlockSpec((NR, NBASis), lambda i: (0, 0)),
            pl.BlockSpec((NBASis, H * H), lambda i: (0, 0)),
        ],
        out_specs=[
            pl.BlockSpec((NBLK, H), lambda i: (i, 0)),
            pl.BlockSpec((NR, NBLK, H), lambda i: (0, i, 0)),
        ],
        out_shape=[
            jax.ShapeDtypeStruct((N, H), jnp.float32),
            jax.ShapeDtypeStruct((NR, N, H), jnp.float32),
        ],
    )(parts, x, root, bias2d, comp, basis_flat)


def kernel(edge_index, edge_type, node_emb, basis1, comp1, root1, bias1,
           basis2, comp2, root2, bias2):
    src = edge_index[0]
    dst = edge_index[1]
    rel = edge_type

    cnt_parts = _counts_call(dst, rel)
    p0 = cnt_parts[:CTAB]
    p1 = cnt_parts[CTAB:]

    b1f = basis1.reshape(NBASis, H * H)
    b2f = basis2.reshape(NBASis, H * H)

    h1 = _transform(node_emb, comp1, b1f)
    parts1 = _agg_call(h1.reshape(NR * N, H), src, rel, dst, p0, p1)
    x1, h2 = _combine_transform(parts1.reshape(NC, N_PAD, H), node_emb,
                                root1, bias1.reshape(1, H), comp2, b2f)
    parts2 = _agg_call(h2.reshape(NR * N, H), src, rel, dst, p0, p1)
    return _combine(parts2.reshape(NC, N_PAD, H), x1, root2,
                    bias2.reshape(1, H), relu=False)
